# baseline (speedup 1.0000x reference)
import functools

import jax
import jax.numpy as jnp
from jax import lax
from jax.experimental import pallas as pl
from jax.experimental.pallas import tpu as pltpu
from jax.experimental.pallas import tpu_sc as plsc

_VB = 10000
_C = 40
_L2E = 1.4426950408889634
_LN2 = 0.6931471805599453
_BIG = 2 ** 30

_NC = 2
_NS = 16
_ROWS_PER_W = 8


def _tc_kern(lt_ref, lp_ref, md_ref, m_ref, i_ref, s_ref):
    j = pl.program_id(0)
    nb = pl.num_programs(0)
    sub = jax.lax.broadcasted_iota(jnp.int32, (_C, 128), 0)

    @pl.when(j == 0)
    def _init():
        m_ref[...] = jnp.full_like(m_ref, -jnp.inf)
        i_ref[...] = jnp.zeros_like(i_ref)
        s_ref[...] = jnp.zeros_like(s_ref)

    m_old = m_ref[...]
    m_acc = m_old
    i_acc = i_ref[...]
    base = j * _VB

    for c in range(_VB // _C):
        x = lt_ref[pl.ds(c * _C, _C), :]
        gt = x > m_acc
        i_acc = jnp.where(gt, jnp.int32(base + c * _C), i_acc)
        m_acc = jnp.maximum(x, m_acc)

    s_acc = s_ref[...] * jnp.exp2((m_old - m_acc) * _L2E)
    m2 = m_acc * _L2E

    for c in range(_VB // _C):
        x = lt_ref[pl.ds(c * _C, _C), :]
        s_acc = s_acc + jnp.exp2(x * _L2E - m2)

    m_ref[...] = m_acc
    i_ref[...] = i_acc
    s_ref[...] = s_acc

    @pl.when(j == nb - 1)
    def _fin():
        m_f = jnp.max(m_acc, axis=0, keepdims=True)
        vi = i_acc + sub
        i_f = jnp.min(jnp.where(m_acc == m_f, vi, _BIG), axis=0,
                      keepdims=True)
        s_f = jnp.sum(s_acc * jnp.exp2((m_acc - m_f) * _L2E), axis=0,
                      keepdims=True)
        lp_ref[...] = -(m_f + _LN2 * jnp.log2(s_f))
        md_ref[...] = i_f


def _neg_lse_and_mode(lt, b):
    nb = lt.shape[0] // _VB
    return pl.pallas_call(
        _tc_kern,
        grid=(nb,),
        in_specs=[pl.BlockSpec((_VB, b), lambda j: (j, 0))],
        out_specs=[
            pl.BlockSpec((1, b), lambda j: (0, 0)),
            pl.BlockSpec((1, b), lambda j: (0, 0)),
        ],
        out_shape=[
            jax.ShapeDtypeStruct((1, b), jnp.float32),
            jax.ShapeDtypeStruct((1, b), jnp.int32),
        ],
        scratch_shapes=[
            pltpu.VMEM((_C, b), jnp.float32),
            pltpu.VMEM((_C, b), jnp.int32),
            pltpu.VMEM((_C, b), jnp.float32),
        ],
    )(lt)


def _sc_kern(lt_hbm, a_hbm, out_hbm, idx_v, rows_v, sem):
    wid = lax.axis_index("s") * _NC + lax.axis_index("c")

    @pl.when(wid < 128 // _ROWS_PER_W)
    def _():
        base = wid * _ROWS_PER_W
        pltpu.sync_copy(a_hbm.at[pl.ds(base, _ROWS_PER_W)], idx_v)
        pltpu.async_copy(lt_hbm.at[idx_v], rows_v, sem).wait()
        pltpu.sync_copy(rows_v, out_hbm.at[pl.ds(base, _ROWS_PER_W)])


def _gather_rows(lt, av):
    b = av.shape[0]
    mesh = plsc.VectorSubcoreMesh(core_axis_name="c", subcore_axis_name="s",
                                  num_cores=_NC, num_subcores=_NS)
    return pl.kernel(
        _sc_kern,
        out_type=jax.ShapeDtypeStruct((b, b), jnp.float32),
        mesh=mesh,
        scratch_types=[
            pltpu.VMEM((_ROWS_PER_W,), jnp.int32),
            pltpu.VMEM((_ROWS_PER_W, b), jnp.float32),
            pltpu.SemaphoreType.DMA,
        ],
    )(lt, av)


@jax.jit
def kernel(logits, actions):
    b, n = logits.shape
    lt = logits.T
    av = actions.reshape(b).astype(jnp.int32)
    rows = _gather_rows(lt, av)
    neg_lse, mode = _neg_lse_and_mode(lt, b)
    g = jnp.take_along_axis(rows, jnp.arange(b)[:, None], axis=1)
    lp = g + neg_lse.reshape(b, 1)
    return lp, mode.reshape(b, 1)


# speedup vs baseline: 1.2802x; 1.2802x over previous
import jax
import jax.numpy as jnp
from jax.experimental import pallas as pl
from jax.experimental.pallas import tpu as pltpu

_VB = 10000
_C = 40
_L2E = 1.4426950408889634
_LN2 = 0.6931471805599453
_BIG = 2 ** 30


def _kern(hits_ref, a_ref, lt_ref, lp_ref, md_ref, m_ref, i_ref, s_ref,
          g_ref):
    j = pl.program_id(0)
    nb = pl.num_programs(0)
    nch = _VB // _C
    sub = jax.lax.broadcasted_iota(jnp.int32, (_C, 128), 0)
    a = a_ref[...]

    @pl.when(j == 0)
    def _init():
        m_ref[...] = jnp.full_like(m_ref, -jnp.inf)
        i_ref[...] = jnp.zeros_like(i_ref)
        s_ref[...] = jnp.zeros_like(s_ref)
        g_ref[...] = jnp.zeros_like(g_ref)

    m_old = m_ref[...]
    m_acc = m_old
    i_acc = i_ref[...]
    base = j * _VB

    for c in range(nch):
        o = base + c * _C
        x = lt_ref[pl.ds(c * _C, _C), :]
        gt = x > m_acc
        i_acc = jnp.where(gt, jnp.int32(o), i_acc)
        m_acc = jnp.maximum(x, m_acc)

        @pl.when(hits_ref[0, j * nch + c] != 0)
        def _gather(x=x, o=o):
            g_ref[...] = g_ref[...] + jnp.where(sub == (a - o), x, 0.0)

    s_acc = s_ref[...] * jnp.exp2((m_old - m_acc) * _L2E)
    m2 = m_acc * _L2E

    for c in range(nch):
        x = lt_ref[pl.ds(c * _C, _C), :]
        s_acc = s_acc + jnp.exp2(x * _L2E - m2)

    m_ref[...] = m_acc
    i_ref[...] = i_acc
    s_ref[...] = s_acc

    @pl.when(j == nb - 1)
    def _fin():
        m_f = jnp.max(m_acc, axis=0, keepdims=True)
        vi = i_acc + sub
        i_f = jnp.min(jnp.where(m_acc == m_f, vi, _BIG), axis=0,
                      keepdims=True)
        s_f = jnp.sum(s_acc * jnp.exp2((m_acc - m_f) * _L2E), axis=0,
                      keepdims=True)
        g_f = jnp.sum(g_ref[...], axis=0, keepdims=True)
        lp_ref[...] = g_f - (m_f + _LN2 * jnp.log2(s_f))
        md_ref[...] = i_f


@jax.jit
def kernel(logits, actions):
    b, n = logits.shape
    lt = logits.T
    a1 = actions.reshape(b).astype(jnp.int32)
    av = a1.reshape(1, b)
    nchunks = n // _C
    hits = jnp.zeros((1, nchunks), jnp.int32).at[0, a1 // _C].set(1)
    nb = n // _VB
    lp, mode = pl.pallas_call(
        _kern,
        grid=(nb,),
        in_specs=[
            pl.BlockSpec((1, nchunks), lambda j: (0, 0),
                         memory_space=pltpu.SMEM),
            pl.BlockSpec((1, b), lambda j: (0, 0)),
            pl.BlockSpec((_VB, b), lambda j: (j, 0)),
        ],
        out_specs=[
            pl.BlockSpec((1, b), lambda j: (0, 0)),
            pl.BlockSpec((1, b), lambda j: (0, 0)),
        ],
        out_shape=[
            jax.ShapeDtypeStruct((1, b), jnp.float32),
            jax.ShapeDtypeStruct((1, b), jnp.int32),
        ],
        scratch_shapes=[
            pltpu.VMEM((_C, b), jnp.float32),
            pltpu.VMEM((_C, b), jnp.int32),
            pltpu.VMEM((_C, b), jnp.float32),
            pltpu.VMEM((_C, b), jnp.float32),
        ],
    )(hits, av, lt)
    return lp.reshape(b, 1), mode.reshape(b, 1)


# speedup vs baseline: 1.3259x; 1.0357x over previous
import jax
import jax.numpy as jnp
from jax.experimental import pallas as pl
from jax.experimental.pallas import tpu as pltpu

_VB = 20000
_C = 40
_L2E = 1.4426950408889634
_LN2 = 0.6931471805599453
_BIG = 2 ** 30


def _kern(a_ref, lt_ref, lp_ref, md_ref, m_ref, i_ref, s_ref, g_ref):
    j = pl.program_id(0)
    nb = pl.num_programs(0)
    sub = jax.lax.broadcasted_iota(jnp.int32, (_C, 128), 0)
    a = a_ref[...]

    @pl.when(j == 0)
    def _init():
        m_ref[...] = jnp.full_like(m_ref, -jnp.inf)
        i_ref[...] = jnp.zeros_like(i_ref)
        s_ref[...] = jnp.zeros_like(s_ref)
        g_ref[...] = jnp.zeros_like(g_ref)

    m_old = m_ref[...]
    m_acc = m_old
    i_acc = i_ref[...]
    g_acc = g_ref[...]
    base = j * _VB

    for c in range(_VB // _C):
        o = base + c * _C
        x = lt_ref[pl.ds(c * _C, _C), :]
        gt = x > m_acc
        i_acc = jnp.where(gt, jnp.int32(o), i_acc)
        m_acc = jnp.maximum(x, m_acc)
        g_acc = g_acc + jnp.where(sub == (a - o), x, 0.0)

    s_acc = s_ref[...] * jnp.exp2((m_old - m_acc) * _L2E)
    m2 = m_acc * _L2E

    for c in range(_VB // _C):
        x = lt_ref[pl.ds(c * _C, _C), :]
        s_acc = s_acc + jnp.exp2(x * _L2E - m2)

    m_ref[...] = m_acc
    i_ref[...] = i_acc
    s_ref[...] = s_acc
    g_ref[...] = g_acc

    @pl.when(j == nb - 1)
    def _fin():
        m_f = jnp.max(m_acc, axis=0, keepdims=True)
        vi = i_acc + sub
        i_f = jnp.min(jnp.where(m_acc == m_f, vi, _BIG), axis=0,
                      keepdims=True)
        s_f = jnp.sum(s_acc * jnp.exp2((m_acc - m_f) * _L2E), axis=0,
                      keepdims=True)
        g_f = jnp.sum(g_acc, axis=0, keepdims=True)
        lp_ref[...] = g_f - (m_f + _LN2 * jnp.log2(s_f))
        md_ref[...] = i_f


@jax.jit
def kernel(logits, actions):
    b, n = logits.shape
    lt = logits.T
    av = actions.reshape(1, b).astype(jnp.int32)
    nb = n // _VB
    lp, mode = pl.pallas_call(
        _kern,
        grid=(nb,),
        in_specs=[
            pl.BlockSpec((1, b), lambda j: (0, 0)),
            pl.BlockSpec((_VB, b), lambda j: (j, 0)),
        ],
        out_specs=[
            pl.BlockSpec((1, b), lambda j: (0, 0)),
            pl.BlockSpec((1, b), lambda j: (0, 0)),
        ],
        out_shape=[
            jax.ShapeDtypeStruct((1, b), jnp.float32),
            jax.ShapeDtypeStruct((1, b), jnp.int32),
        ],
        scratch_shapes=[
            pltpu.VMEM((_C, b), jnp.float32),
            pltpu.VMEM((_C, b), jnp.int32),
            pltpu.VMEM((_C, b), jnp.float32),
            pltpu.VMEM((_C, b), jnp.float32),
        ],
    )(av, lt)
    return lp.reshape(b, 1), mode.reshape(b, 1)


# speedup vs baseline: 1.3812x; 1.0417x over previous
import jax
import jax.numpy as jnp
from jax.experimental import pallas as pl
from jax.experimental.pallas import tpu as pltpu

_VB = 10000
_C = 40
_L2E = 1.4426950408889634
_LN2 = 0.6931471805599453
_BIG = 2 ** 30


def _kern(a_ref, lt_ref, lp_ref, md_ref, m_ref, i_ref, s_ref, g_ref):
    j = pl.program_id(0)
    nb = pl.num_programs(0)
    sub = jax.lax.broadcasted_iota(jnp.int32, (_C, 128), 0)
    a = a_ref[...]

    @pl.when(j == 0)
    def _init():
        m_ref[...] = jnp.full_like(m_ref, -jnp.inf)
        i_ref[...] = jnp.zeros_like(i_ref)
        s_ref[...] = jnp.zeros_like(s_ref)
        g_ref[...] = jnp.zeros_like(g_ref)

    m_old = m_ref[...]
    m_acc = m_old
    i_acc = i_ref[...]
    g_acc = g_ref[...]
    base = j * _VB

    for c in range(_VB // _C):
        o = base + c * _C
        x = lt_ref[pl.ds(c * _C, _C), :]
        gt = x > m_acc
        i_acc = jnp.where(gt, jnp.int32(o), i_acc)
        m_acc = jnp.maximum(x, m_acc)
        g_acc = g_acc + jnp.where(sub == (a - o), x, 0.0)

    s_acc = s_ref[...] * jnp.exp2((m_old - m_acc) * _L2E)
    m2 = m_acc * _L2E

    for c in range(_VB // _C):
        x = lt_ref[pl.ds(c * _C, _C), :]
        s_acc = s_acc + jnp.exp2(x * _L2E - m2)

    m_ref[...] = m_acc
    i_ref[...] = i_acc
    s_ref[...] = s_acc
    g_ref[...] = g_acc

    @pl.when(j == nb - 1)
    def _fin():
        m_f = jnp.max(m_acc, axis=0, keepdims=True)
        vi = i_acc + sub
        i_f = jnp.min(jnp.where(m_acc == m_f, vi, _BIG), axis=0,
                      keepdims=True)
        s_f = jnp.sum(s_acc * jnp.exp2((m_acc - m_f) * _L2E), axis=0,
                      keepdims=True)
        g_f = jnp.sum(g_acc, axis=0, keepdims=True)
        lp_ref[...] = g_f - (m_f + _LN2 * jnp.log2(s_f))
        md_ref[...] = i_f


@jax.jit
def kernel(logits, actions):
    b, n = logits.shape
    lt = logits.T
    av = actions.reshape(1, b).astype(jnp.int32)
    nb = n // _VB
    lp, mode = pl.pallas_call(
        _kern,
        grid=(nb,),
        in_specs=[
            pl.BlockSpec((1, b), lambda j: (0, 0)),
            pl.BlockSpec((_VB, b), lambda j: (j, 0)),
        ],
        out_specs=[
            pl.BlockSpec((1, b), lambda j: (0, 0)),
            pl.BlockSpec((1, b), lambda j: (0, 0)),
        ],
        out_shape=[
            jax.ShapeDtypeStruct((1, b), jnp.float32),
            jax.ShapeDtypeStruct((1, b), jnp.int32),
        ],
        scratch_shapes=[
            pltpu.VMEM((_C, b), jnp.float32),
            pltpu.VMEM((_C, b), jnp.int32),
            pltpu.VMEM((_C, b), jnp.float32),
            pltpu.VMEM((_C, b), jnp.float32),
        ],
    )(av, lt)
    return lp.reshape(b, 1), mode.reshape(b, 1)
